# baseline (speedup 1.0000x reference)
import jax
import jax.numpy as jnp
from jax import lax
from jax.experimental import pallas as pl
from jax.experimental.pallas import tpu as pltpu
from jax.experimental.pallas import tpu_sc as plsc

_C = 1024
_S = 2048
_L = 4096
_NC = 2
_NS = 16
_NW = _NC * _NS
_BPW = _S // _NW
_CROWS = _C // _NW
_TCOL = 16


_CCH = 8


def _body(x_hbm, cache_hbm, emb_hbm, out_hbm, idx_v, rows_v, trans_v, cbuf,
          gsem):
    wid = lax.axis_index("s") * _NC + lax.axis_index("c")
    base = wid * _BPW

    pltpu.sync_copy(x_hbm.at[pl.ds(base, _BPW)], idx_v)
    pltpu.async_copy(emb_hbm.at[idx_v], rows_v, gsem).wait()

    for t in range(_BPW // _TCOL):
        j_idx = t * _TCOL + lax.iota(jnp.int32, _TCOL)

        def body(c, carry, j_idx=j_idx):
            vals = plsc.load_gather(
                rows_v, [j_idx, jnp.full((_TCOL,), c, jnp.int32)])
            trans_v[c] = vals
            return carry

        lax.fori_loop(0, _C, body, 0)
        pltpu.sync_copy(
            trans_v, out_hbm.at[:, pl.ds(_S + base + t * _TCOL, _TCOL)])

    crow = wid * _CROWS
    for k in range(_CROWS // _CCH):
        pltpu.sync_copy(
            cache_hbm.at[pl.ds(crow + k * _CCH, _CCH), pl.ds(_S, _L - _S)],
            cbuf)
        pltpu.sync_copy(
            cbuf,
            out_hbm.at[pl.ds(crow + k * _CCH, _CCH), pl.ds(0, _L - _S)])


@jax.jit
def _net(x_flat, cache2d, emb):
    mesh = plsc.VectorSubcoreMesh(core_axis_name="c", subcore_axis_name="s")
    return pl.kernel(
        _body,
        out_type=jax.ShapeDtypeStruct((_C, _L), jnp.float32),
        mesh=mesh,
        compiler_params=pltpu.CompilerParams(
            use_tc_tiling_on_sc=False, needs_layout_passes=False),
        scratch_types=[
            pltpu.VMEM((_BPW,), jnp.int32),
            pltpu.VMEM((_BPW, _C), jnp.float32),
            pltpu.VMEM((_C, _TCOL), jnp.float32),
            pltpu.VMEM((_CCH, _L - _S), jnp.float32),
            pltpu.SemaphoreType.DMA,
        ],
    )(x_flat, cache2d, emb)


def kernel(x, cache, emb):
    out2d = _net(x.reshape(_S), cache.reshape(_C, _L), emb)
    return out2d.reshape(1, _C, 1, _L)


# speedup vs baseline: 1.0086x; 1.0086x over previous
import jax
import jax.numpy as jnp
from jax import lax
from jax.experimental import pallas as pl
from jax.experimental.pallas import tpu as pltpu
from jax.experimental.pallas import tpu_sc as plsc

_C = 1024
_S = 2048
_L = 4096
_NC = 2
_NS = 16
_NW = _NC * _NS
_NBLK = 16
_IPB = _S // _NBLK
_CHALF = _C // 2
_GCH = 64
_CROWS = _C // _NW
_CCH = 8


def _body(x_hbm, cache_hbm, emb_hbm, out_hbm, idx_v, rows_v, trans_v, cbuf,
          gsem):
    wid = lax.axis_index("s") * _NC + lax.axis_index("c")
    blk = wid // 2
    half = wid % 2
    coff = half * _CHALF

    pltpu.sync_copy(x_hbm.at[pl.ds(blk * _IPB, _IPB)], idx_v)

    for g in range(_IPB // _GCH):
        pltpu.async_copy(
            emb_hbm.at[idx_v.at[pl.ds(g * _GCH, _GCH)], pl.ds(coff, _CHALF)],
            rows_v, gsem).wait()
        for jj in range(_GCH // 16):
            j_idx = jj * 16 + lax.iota(jnp.int32, 16)

            def body(c, carry, j_idx=j_idx, g=g, jj=jj):
                vals = plsc.load_gather(
                    rows_v, [j_idx, jnp.full((16,), c, jnp.int32)])
                trans_v[c, pl.ds(g * _GCH + jj * 16, 16)] = vals
                return carry

            lax.fori_loop(0, _CHALF, body, 0, unroll=4)

    pltpu.sync_copy(
        trans_v, out_hbm.at[pl.ds(coff, _CHALF), pl.ds(_S + blk * _IPB, _IPB)])

    crow = wid * _CROWS
    for k in range(_CROWS // _CCH):
        pltpu.sync_copy(
            cache_hbm.at[pl.ds(crow + k * _CCH, _CCH), pl.ds(_S, _L - _S)],
            cbuf)
        pltpu.sync_copy(
            cbuf,
            out_hbm.at[pl.ds(crow + k * _CCH, _CCH), pl.ds(0, _L - _S)])


@jax.jit
def _net(x_flat, cache2d, emb):
    mesh = plsc.VectorSubcoreMesh(core_axis_name="c", subcore_axis_name="s")
    return pl.kernel(
        _body,
        out_type=jax.ShapeDtypeStruct((_C, _L), jnp.float32),
        mesh=mesh,
        compiler_params=pltpu.CompilerParams(needs_layout_passes=False),
        scratch_types=[
            pltpu.VMEM((_IPB,), jnp.int32),
            pltpu.VMEM((_GCH, _CHALF), jnp.float32),
            pltpu.VMEM((_CHALF, _IPB), jnp.float32),
            pltpu.VMEM((_CCH, _L - _S), jnp.float32),
            pltpu.SemaphoreType.DMA,
        ],
    )(x_flat, cache2d, emb)


def kernel(x, cache, emb):
    out2d = _net(x.reshape(_S), cache.reshape(_C, _L), emb)
    return out2d.reshape(1, _C, 1, _L)


# speedup vs baseline: 2.2501x; 2.2310x over previous
import jax
import jax.numpy as jnp
from jax import lax
from jax.experimental import pallas as pl
from jax.experimental.pallas import tpu as pltpu
from jax.experimental.pallas import tpu_sc as plsc

_C = 1024
_S = 2048
_L = 4096
_NC = 2
_NS = 16
_NW = _NC * _NS
_NBLK = 16
_IPB = _S // _NBLK
_CHALF = _C // 2
_CROWS = _C // _NW
_RPC = _L // 128
_SH = _S // 128
_NCCH = 8


def _body(x_hbm, cache_hbm, emb_hbm, out_hbm, idx_v, rows_v,
          trans_a, trans_b, cbuf_a, cbuf_b, tidx_a, tidx_b,
          gidx_a, gidx_b, sidx_a, sidx_b, gsem, tsem, cgsem, cssem):
    wid = lax.axis_index("s") * _NC + lax.axis_index("c")
    blk = wid // 2
    half = wid % 2
    coff = half * _CHALF
    crow = wid * _CROWS
    iota = lax.iota(jnp.int32, 16)
    trans = (trans_a, trans_b)
    cbuf = (cbuf_a, cbuf_b)
    tidx = (tidx_a, tidx_b)
    gidx = (gidx_a, gidx_b)
    sidx = (sidx_a, sidx_b)

    pltpu.sync_copy(x_hbm.at[pl.ds(blk * _IPB, _IPB)], idx_v)
    h_emb = pltpu.async_copy(
        emb_hbm.at[idx_v, pl.ds(coff, _CHALF)], rows_v, gsem)

    def cache_fire_gather(k):
        s = k % 2
        for cc in range(4):
            base = (crow + k * 4 + cc) * _RPC
            gidx[s][pl.ds(cc * 16, 16)] = base + _SH + iota
            sidx[s][pl.ds(cc * 16, 16)] = base + iota
        return pltpu.async_copy(cache_hbm.at[gidx[s]], cbuf[s], cgsem)

    h_cg = [None] * _NCCH
    h_cs = [None] * _NCCH
    h_cg[0] = cache_fire_gather(0)

    h_emb.wait()

    h_ts = [None] * 4
    for q in range(4):
        s = q % 2
        if q >= 2:
            h_ts[q - 2].wait()

        @plsc.parallel_loop(0, 16, unroll=2)
        def tq(k, q=q, s=s):
            rot = jnp.bitwise_and(iota + k, 15)
            for ct in range(8):
                c_idx = ct * 16 + iota
                for jt in range(8):
                    j_idx = jt * 16 + rot
                    vals = plsc.load_gather(
                        rows_v, [j_idx, q * 128 + c_idx])
                    plsc.store_scatter(trans[s], [c_idx, j_idx], vals)

        for v in range(8):
            tidx[s][pl.ds(v * 16, 16)] = (
                (coff + q * 128 + v * 16 + iota) * _RPC + _SH + blk)
        h_ts[q] = pltpu.async_copy(trans[s], out_hbm.at[tidx[s]], tsem)

        for k in (2 * q, 2 * q + 1):
            h_cg[k].wait()
            h_cs[k] = pltpu.async_copy(
                cbuf[k % 2], out_hbm.at[sidx[k % 2]], cssem)
            if k + 1 < _NCCH:
                if k >= 1:
                    h_cs[k - 1].wait()
                h_cg[k + 1] = cache_fire_gather(k + 1)

    h_ts[2].wait()
    h_ts[3].wait()
    h_cs[_NCCH - 2].wait()
    h_cs[_NCCH - 1].wait()


@jax.jit
def _net(x_flat, cache2d, emb):
    mesh = plsc.VectorSubcoreMesh(core_axis_name="c", subcore_axis_name="s")
    return pl.kernel(
        _body,
        out_type=jax.ShapeDtypeStruct((_C * _L // 128, 128), jnp.float32),
        mesh=mesh,
        compiler_params=pltpu.CompilerParams(needs_layout_passes=False),
        scratch_types=[
            pltpu.VMEM((_IPB,), jnp.int32),
            pltpu.VMEM((_IPB, _CHALF), jnp.float32),
            pltpu.VMEM((128, 128), jnp.float32),
            pltpu.VMEM((128, 128), jnp.float32),
            pltpu.VMEM((64, 128), jnp.float32),
            pltpu.VMEM((64, 128), jnp.float32),
            pltpu.VMEM((128,), jnp.int32),
            pltpu.VMEM((128,), jnp.int32),
            pltpu.VMEM((64,), jnp.int32),
            pltpu.VMEM((64,), jnp.int32),
            pltpu.VMEM((64,), jnp.int32),
            pltpu.VMEM((64,), jnp.int32),
            pltpu.SemaphoreType.DMA,
            pltpu.SemaphoreType.DMA,
            pltpu.SemaphoreType.DMA,
            pltpu.SemaphoreType.DMA,
        ],
    )(x_flat, cache2d, emb)


def kernel(x, cache, emb):
    out2d = _net(x.reshape(_S), cache.reshape(_C * _L // 128, 128), emb)
    return out2d.reshape(1, _C, 1, _L)


# speedup vs baseline: 2.3741x; 1.0551x over previous
import jax
import jax.numpy as jnp
from jax import lax
from jax.experimental import pallas as pl
from jax.experimental.pallas import tpu as pltpu
from jax.experimental.pallas import tpu_sc as plsc

_C = 1024
_S = 2048
_L = 4096
_NC = 2
_NS = 16
_NW = _NC * _NS
_NBLK = 16
_IPB = _S // _NBLK
_CHALF = _C // 2
_CROWS = _C // _NW
_RPC = _L // 128
_SH = _S // 128
_NCCH = 8


def _body(x_hbm, cache_hbm, emb_hbm, out_hbm, idx_v, rows_a, rows_b,
          trans_a, trans_b, cbuf_a, cbuf_b, tidx_a, tidx_b,
          gidx_a, gidx_b, sidx_a, sidx_b, gsem, tsem, cgsem, cssem):
    wid = lax.axis_index("s") * _NC + lax.axis_index("c")
    blk = wid // 2
    half = wid % 2
    coff = half * _CHALF
    crow = wid * _CROWS
    iota = lax.iota(jnp.int32, 16)
    rows = (rows_a, rows_b)
    trans = (trans_a, trans_b)
    cbuf = (cbuf_a, cbuf_b)
    tidx = (tidx_a, tidx_b)
    gidx = (gidx_a, gidx_b)
    sidx = (sidx_a, sidx_b)

    pltpu.sync_copy(x_hbm.at[pl.ds(blk * _IPB, _IPB)], idx_v)
    h_emb = [None, None]
    h_emb[0] = pltpu.async_copy(
        emb_hbm.at[idx_v, pl.ds(coff, _CHALF // 2)], rows_a, gsem)

    def cache_fire_gather(k):
        s = k % 2
        for cc in range(4):
            base = (crow + k * 4 + cc) * _RPC
            gidx[s][pl.ds(cc * 16, 16)] = base + _SH + iota
            sidx[s][pl.ds(cc * 16, 16)] = base + iota
        return pltpu.async_copy(cache_hbm.at[gidx[s]], cbuf[s], cgsem)

    h_cg = [None] * _NCCH
    h_cs = [None] * _NCCH
    h_cg[0] = cache_fire_gather(0)
    h_emb[1] = pltpu.async_copy(
        emb_hbm.at[idx_v, pl.ds(coff + _CHALF // 2, _CHALF // 2)],
        rows_b, gsem)

    def do_cache_step(k):
        h_cg[k].wait()
        h_cs[k] = pltpu.async_copy(
            cbuf[k % 2], out_hbm.at[sidx[k % 2]], cssem)
        if k + 1 < _NCCH:
            if k >= 1:
                h_cs[k - 1].wait()
            h_cg[k + 1] = cache_fire_gather(k + 1)

    do_cache_step(0)

    cache_sched = {0: (1,), 1: (2, 3), 2: (4, 5), 3: (6, 7)}

    h_ts = [None] * 4
    for q in range(4):
        s = q % 2
        if q in (0, 2):
            h_emb[q // 2].wait()
        if q >= 2:
            h_ts[q - 2].wait()

        @plsc.parallel_loop(0, 16, unroll=2)
        def tq(k, q=q, s=s):
            rot = jnp.bitwise_and(iota + k, 15)
            for ct in range(8):
                c_idx = ct * 16 + iota
                for jt in range(8):
                    j_idx = jt * 16 + rot
                    vals = plsc.load_gather(
                        rows[q // 2], [j_idx, (q % 2) * 128 + c_idx])
                    plsc.store_scatter(trans[s], [c_idx, j_idx], vals)

        for v in range(8):
            tidx[s][pl.ds(v * 16, 16)] = (
                (coff + q * 128 + v * 16 + iota) * _RPC + _SH + blk)
        h_ts[q] = pltpu.async_copy(trans[s], out_hbm.at[tidx[s]], tsem)

        for k in cache_sched[q]:
            do_cache_step(k)

    h_ts[2].wait()
    h_ts[3].wait()
    h_cs[_NCCH - 2].wait()
    h_cs[_NCCH - 1].wait()


@jax.jit
def _net(x_flat, cache2d, emb):
    mesh = plsc.VectorSubcoreMesh(core_axis_name="c", subcore_axis_name="s")
    return pl.kernel(
        _body,
        out_type=jax.ShapeDtypeStruct((_C * _L // 128, 128), jnp.float32),
        mesh=mesh,
        compiler_params=pltpu.CompilerParams(needs_layout_passes=False),
        scratch_types=[
            pltpu.VMEM((_IPB,), jnp.int32),
            pltpu.VMEM((_IPB, _CHALF // 2), jnp.float32),
            pltpu.VMEM((_IPB, _CHALF // 2), jnp.float32),
            pltpu.VMEM((128, 128), jnp.float32),
            pltpu.VMEM((128, 128), jnp.float32),
            pltpu.VMEM((64, 128), jnp.float32),
            pltpu.VMEM((64, 128), jnp.float32),
            pltpu.VMEM((128,), jnp.int32),
            pltpu.VMEM((128,), jnp.int32),
            pltpu.VMEM((64,), jnp.int32),
            pltpu.VMEM((64,), jnp.int32),
            pltpu.VMEM((64,), jnp.int32),
            pltpu.VMEM((64,), jnp.int32),
            pltpu.SemaphoreType.DMA,
            pltpu.SemaphoreType.DMA,
            pltpu.SemaphoreType.DMA,
            pltpu.SemaphoreType.DMA,
        ],
    )(x_flat, cache2d, emb)


def kernel(x, cache, emb):
    out2d = _net(x.reshape(_S), cache.reshape(_C * _L // 128, 128), emb)
    return out2d.reshape(1, _C, 1, _L)


# speedup vs baseline: 2.5145x; 1.0591x over previous
import jax
import jax.numpy as jnp
from jax import lax
from jax.experimental import pallas as pl
from jax.experimental.pallas import tpu as pltpu
from jax.experimental.pallas import tpu_sc as plsc

_C = 1024
_S = 2048
_L = 4096
_NC = 2
_NS = 16
_NW = _NC * _NS
_NBLK = 16
_IPB = _S // _NBLK
_CHALF = _C // 2
_CROWS = _C // _NW
_RPC = _L // 128
_SH = _S // 128
_NCCH = 8


def _body(x_hbm, cache_hbm, emb_hbm, out_hbm, idx_v, rows_a, rows_b,
          trans_a, trans_b, cbuf_a, cbuf_b, tidx_a, tidx_b,
          gidx_a, gidx_b, sidx_a, sidx_b, gsem, tsem, cgsem, cssem):
    wid = lax.axis_index("s") * _NC + lax.axis_index("c")
    blk = wid // 2
    half = wid % 2
    coff = half * _CHALF
    crow = wid * _CROWS
    iota = lax.iota(jnp.int32, 16)
    rows = (rows_a, rows_b)
    trans = (trans_a, trans_b)
    cbuf = (cbuf_a, cbuf_b)
    tidx = (tidx_a, tidx_b)
    gidx = (gidx_a, gidx_b)
    sidx = (sidx_a, sidx_b)

    pltpu.sync_copy(x_hbm.at[pl.ds(blk * _IPB, _IPB)], idx_v)
    h_emb = [None, None]
    h_emb[0] = pltpu.async_copy(
        emb_hbm.at[idx_v, pl.ds(coff, _CHALF // 2)], rows_a, gsem)

    def cache_fire_gather(k):
        s = k % 2
        for cc in range(4):
            base = (crow + k * 4 + cc) * _RPC
            gidx[s][pl.ds(cc * 16, 16)] = base + _SH + iota
            sidx[s][pl.ds(cc * 16, 16)] = base + iota
        return pltpu.async_copy(cache_hbm.at[gidx[s]], cbuf[s], cgsem)

    h_cg = [None] * _NCCH
    h_cs = [None] * _NCCH
    h_cg[0] = cache_fire_gather(0)
    h_emb[1] = pltpu.async_copy(
        emb_hbm.at[idx_v, pl.ds(coff + _CHALF // 2, _CHALF // 2)],
        rows_b, gsem)

    def do_cache_step(k):
        h_cg[k].wait()
        h_cs[k] = pltpu.async_copy(
            cbuf[k % 2], out_hbm.at[sidx[k % 2]], cssem)
        if k + 1 < _NCCH:
            if k >= 1:
                h_cs[k - 1].wait()
            h_cg[k + 1] = cache_fire_gather(k + 1)

    do_cache_step(0)

    cache_sched = {0: (1,), 1: (2, 3), 2: (4, 5), 3: (6, 7)}

    h_ts = [None] * 4
    for q in range(4):
        s = q % 2
        if q in (0, 2):
            h_emb[q // 2].wait()
        if q >= 2:
            h_ts[q - 2].wait()

        @plsc.parallel_loop(0, 16)
        def tq(k, q=q, s=s):
            rot = jnp.bitwise_and(iota + k, 15)
            for ct in range(8):
                c_idx = ct * 16 + iota
                for jt in range(8):
                    j_idx = jt * 16 + rot
                    vals = plsc.load_gather(
                        rows[q // 2], [j_idx, (q % 2) * 128 + c_idx])
                    plsc.store_scatter(trans[s], [c_idx, j_idx], vals)

        for v in range(8):
            tidx[s][pl.ds(v * 16, 16)] = (
                (coff + q * 128 + v * 16 + iota) * _RPC + _SH + blk)
        h_ts[q] = pltpu.async_copy(trans[s], out_hbm.at[tidx[s]], tsem)

        for k in cache_sched[q]:
            do_cache_step(k)

    h_ts[2].wait()
    h_ts[3].wait()
    h_cs[_NCCH - 2].wait()
    h_cs[_NCCH - 1].wait()


@jax.jit
def _net(x_flat, cache2d, emb):
    mesh = plsc.VectorSubcoreMesh(core_axis_name="c", subcore_axis_name="s")
    return pl.kernel(
        _body,
        out_type=jax.ShapeDtypeStruct((_C * _L // 128, 128), jnp.float32),
        mesh=mesh,
        compiler_params=pltpu.CompilerParams(needs_layout_passes=False),
        scratch_types=[
            pltpu.VMEM((_IPB,), jnp.int32),
            pltpu.VMEM((_IPB, _CHALF // 2), jnp.float32),
            pltpu.VMEM((_IPB, _CHALF // 2), jnp.float32),
            pltpu.VMEM((128, 128), jnp.float32),
            pltpu.VMEM((128, 128), jnp.float32),
            pltpu.VMEM((64, 128), jnp.float32),
            pltpu.VMEM((64, 128), jnp.float32),
            pltpu.VMEM((128,), jnp.int32),
            pltpu.VMEM((128,), jnp.int32),
            pltpu.VMEM((64,), jnp.int32),
            pltpu.VMEM((64,), jnp.int32),
            pltpu.VMEM((64,), jnp.int32),
            pltpu.VMEM((64,), jnp.int32),
            pltpu.SemaphoreType.DMA,
            pltpu.SemaphoreType.DMA,
            pltpu.SemaphoreType.DMA,
            pltpu.SemaphoreType.DMA,
        ],
    )(x_flat, cache2d, emb)


def kernel(x, cache, emb):
    out2d = _net(x.reshape(_S), cache.reshape(_C * _L // 128, 128), emb)
    return out2d.reshape(1, _C, 1, _L)
